# baseline (speedup 1.0000x reference)
import jax
import jax.numpy as jnp
from jax.experimental import pallas as pl
from jax.experimental.pallas import tpu as pltpu


def _tc_body(lab_ref, x_ref, p_ref, n_ref, out_ref, acc_sum, acc_cnt):
    i = pl.program_id(0)
    nb = pl.num_programs(0)

    @pl.when(i == 0)
    def _():
        acc_sum[...] = jnp.zeros_like(acc_sum)
        acc_cnt[...] = jnp.zeros_like(acc_cnt)

    x = x_ref[0]
    y = lab_ref[0, 0, 0]
    m = jnp.where((p_ref[0] > 0.5) & (n_ref[0] > 0.5), 1.0, 0.0)
    bce = jnp.maximum(x, 0.0) - x * y + jnp.log1p(jnp.exp(-jnp.abs(x)))
    acc_sum[...] += jnp.sum(bce * m, axis=0, keepdims=True)
    acc_cnt[...] += jnp.sum(m, axis=0, keepdims=True)

    @pl.when(i == nb - 1)
    def _():
        out_ref[0, 0] = jnp.sum(acc_sum[...]) / jnp.sum(acc_cnt[...])


def kernel(cancer_logits, label, prostate_mask, needle_mask):
    B, C, H, W = cancer_logits.shape
    x = cancer_logits.reshape(B, H, W)
    p = prostate_mask.reshape(B, H, W)
    n = needle_mask.reshape(B, H, W)
    lab = label.reshape(B, 1, 1)

    out = pl.pallas_call(
        _tc_body,
        grid=(B,),
        in_specs=[
            pl.BlockSpec((1, 1, 1), lambda i: (i, 0, 0), memory_space=pltpu.SMEM),
            pl.BlockSpec((1, H, W), lambda i: (i, 0, 0)),
            pl.BlockSpec((1, H, W), lambda i: (i, 0, 0)),
            pl.BlockSpec((1, H, W), lambda i: (i, 0, 0)),
        ],
        out_specs=pl.BlockSpec((1, 1), lambda i: (0, 0), memory_space=pltpu.SMEM),
        out_shape=jax.ShapeDtypeStruct((1, 1), jnp.float32),
        scratch_shapes=[
            pltpu.VMEM((1, W), jnp.float32),
            pltpu.VMEM((1, W), jnp.float32),
        ],
    )(lab, x, p, n)
    return out[0, 0]


# speedup vs baseline: 1.1454x; 1.1454x over previous
import jax
import jax.numpy as jnp
from jax.experimental import pallas as pl
from jax.experimental.pallas import tpu as pltpu

_LOG2E = 1.4426950408889634
_LN2 = 0.6931471805599453
_R = 16


def _tc_body(lab_ref, x_ref, p_ref, n_ref, out_ref, acc_ref):
    i = pl.program_id(0)
    nb = pl.num_programs(0)

    @pl.when(i == 0)
    def _():
        acc_ref[...] = jnp.zeros_like(acc_ref)

    s = 1.0 - 2.0 * lab_ref[0, 0, 0]
    H = x_ref.shape[1]
    W = x_ref.shape[2]

    def body(j, carry):
        asum, acnt = carry
        x = x_ref[0, pl.ds(j * _R, _R), :]
        p = p_ref[0, pl.ds(j * _R, _R), :]
        n = n_ref[0, pl.ds(j * _R, _R), :]
        mask = (p > 0.5) & (n > 0.5)
        t = x * s
        u = jnp.exp2(jnp.abs(x) * (-_LOG2E))
        bce = jnp.maximum(t, 0.0) + _LN2 * jnp.log2(1.0 + u)
        asum = asum + jnp.where(mask, bce, 0.0)
        acnt = acnt + jnp.where(mask, 1.0, 0.0)
        return asum, acnt

    z = jnp.zeros((_R, W), jnp.float32)
    asum, acnt = jax.lax.fori_loop(0, H // _R, body, (z, z))
    acc_ref[0:1, :] += jnp.sum(asum, axis=0, keepdims=True)
    acc_ref[1:2, :] += jnp.sum(acnt, axis=0, keepdims=True)

    @pl.when(i == nb - 1)
    def _():
        out_ref[0, 0] = jnp.sum(acc_ref[0, :]) / jnp.sum(acc_ref[1, :])


def kernel(cancer_logits, label, prostate_mask, needle_mask):
    B, C, H, W = cancer_logits.shape
    x = cancer_logits.reshape(B, H, W)
    p = prostate_mask.reshape(B, H, W)
    n = needle_mask.reshape(B, H, W)
    lab = label.reshape(B, 1, 1)

    out = pl.pallas_call(
        _tc_body,
        grid=(B,),
        in_specs=[
            pl.BlockSpec((1, 1, 1), lambda i: (i, 0, 0), memory_space=pltpu.SMEM),
            pl.BlockSpec((1, H, W), lambda i: (i, 0, 0)),
            pl.BlockSpec((1, H, W), lambda i: (i, 0, 0)),
            pl.BlockSpec((1, H, W), lambda i: (i, 0, 0)),
        ],
        out_specs=pl.BlockSpec((1, 1), lambda i: (0, 0), memory_space=pltpu.SMEM),
        out_shape=jax.ShapeDtypeStruct((1, 1), jnp.float32),
        scratch_shapes=[
            pltpu.VMEM((2, W), jnp.float32),
        ],
    )(lab, x, p, n)
    return out[0, 0]


# speedup vs baseline: 1.4888x; 1.2998x over previous
import jax
import jax.numpy as jnp
from jax.experimental import pallas as pl
from jax.experimental.pallas import tpu as pltpu

_LOG2E = 1.4426950408889634
_LN2 = 0.6931471805599453
_R = 16
_BC = 2


def _tc_body(lab_ref, x_ref, p_ref, n_ref, out_ref, acc_ref):
    i = pl.program_id(0)
    nb = pl.num_programs(0)

    @pl.when(i == 0)
    def _():
        acc_ref[...] = jnp.zeros_like(acc_ref)

    H = x_ref.shape[1]
    W = x_ref.shape[2]
    ss = [1.0 - 2.0 * lab_ref[b, 0, 0] for b in range(_BC)]

    def body(j, carry):
        asum, acnt = carry
        for b in range(_BC):
            x = x_ref[b, pl.ds(j * _R, _R), :]
            p = p_ref[b, pl.ds(j * _R, _R), :]
            n = n_ref[b, pl.ds(j * _R, _R), :]
            mask = (p > 0.5) & (n > 0.5)
            t = x * ss[b]
            u = jnp.exp2(jnp.abs(x) * (-_LOG2E))
            bce = jnp.maximum(t, 0.0) + _LN2 * jnp.log2(1.0 + u)
            asum = asum + jnp.where(mask, bce, 0.0)
            acnt = acnt + jnp.where(mask, 1.0, 0.0)
        return asum, acnt

    z = jnp.zeros((_R, W), jnp.float32)
    asum, acnt = jax.lax.fori_loop(0, H // _R, body, (z, z))
    acc_ref[0:1, :] += jnp.sum(asum, axis=0, keepdims=True)
    acc_ref[1:2, :] += jnp.sum(acnt, axis=0, keepdims=True)

    @pl.when(i == nb - 1)
    def _():
        out_ref[0, 0] = jnp.sum(acc_ref[0, :]) / jnp.sum(acc_ref[1, :])


def kernel(cancer_logits, label, prostate_mask, needle_mask):
    B, C, H, W = cancer_logits.shape
    x = cancer_logits.reshape(B, H, W)
    p = prostate_mask.reshape(B, H, W)
    n = needle_mask.reshape(B, H, W)
    lab = label.reshape(B, 1, 1)

    out = pl.pallas_call(
        _tc_body,
        grid=(B // _BC,),
        in_specs=[
            pl.BlockSpec((_BC, 1, 1), lambda i: (i, 0, 0), memory_space=pltpu.SMEM),
            pl.BlockSpec((_BC, H, W), lambda i: (i, 0, 0)),
            pl.BlockSpec((_BC, H, W), lambda i: (i, 0, 0)),
            pl.BlockSpec((_BC, H, W), lambda i: (i, 0, 0)),
        ],
        out_specs=pl.BlockSpec((1, 1), lambda i: (0, 0), memory_space=pltpu.SMEM),
        out_shape=jax.ShapeDtypeStruct((1, 1), jnp.float32),
        scratch_shapes=[
            pltpu.VMEM((2, W), jnp.float32),
        ],
    )(lab, x, p, n)
    return out[0, 0]


# speedup vs baseline: 1.6312x; 1.0957x over previous
import jax
import jax.numpy as jnp
from jax.experimental import pallas as pl
from jax.experimental.pallas import tpu as pltpu

_LOG2E = 1.4426950408889634
_LN2 = 0.6931471805599453
_R = 16
_BC = 4


def _tc_body(lab_ref, x_ref, p_ref, n_ref, out_ref, acc_ref):
    i = pl.program_id(0)
    nb = pl.num_programs(0)

    @pl.when(i == 0)
    def _():
        acc_ref[...] = jnp.zeros_like(acc_ref)

    H = x_ref.shape[1]
    W = x_ref.shape[2]
    ss = [1.0 - 2.0 * lab_ref[b, 0, 0] for b in range(_BC)]

    def body(j, carry):
        asum, acnt = carry
        for b in range(_BC):
            x = x_ref[b, pl.ds(j * _R, _R), :]
            p = p_ref[b, pl.ds(j * _R, _R), :]
            n = n_ref[b, pl.ds(j * _R, _R), :]
            mask = (p > 0.5) & (n > 0.5)
            t = x * ss[b]
            u = jnp.exp2(jnp.abs(x) * (-_LOG2E))
            bce = jnp.maximum(t, 0.0) + _LN2 * jnp.log2(1.0 + u)
            asum = asum + jnp.where(mask, bce, 0.0)
            acnt = acnt + jnp.where(mask, 1.0, 0.0)
        return asum, acnt

    z = jnp.zeros((_R, W), jnp.float32)
    asum, acnt = jax.lax.fori_loop(0, H // _R, body, (z, z))
    acc_ref[0:1, :] += jnp.sum(asum, axis=0, keepdims=True)
    acc_ref[1:2, :] += jnp.sum(acnt, axis=0, keepdims=True)

    @pl.when(i == nb - 1)
    def _():
        out_ref[0, 0] = jnp.sum(acc_ref[0, :]) / jnp.sum(acc_ref[1, :])


def kernel(cancer_logits, label, prostate_mask, needle_mask):
    B, C, H, W = cancer_logits.shape
    x = cancer_logits.reshape(B, H, W)
    p = prostate_mask.reshape(B, H, W)
    n = needle_mask.reshape(B, H, W)
    lab = label.reshape(B, 1, 1)

    out = pl.pallas_call(
        _tc_body,
        grid=(B // _BC,),
        in_specs=[
            pl.BlockSpec((_BC, 1, 1), lambda i: (i, 0, 0), memory_space=pltpu.SMEM),
            pl.BlockSpec((_BC, H, W), lambda i: (i, 0, 0)),
            pl.BlockSpec((_BC, H, W), lambda i: (i, 0, 0)),
            pl.BlockSpec((_BC, H, W), lambda i: (i, 0, 0)),
        ],
        out_specs=pl.BlockSpec((1, 1), lambda i: (0, 0), memory_space=pltpu.SMEM),
        out_shape=jax.ShapeDtypeStruct((1, 1), jnp.float32),
        scratch_shapes=[
            pltpu.VMEM((2, W), jnp.float32),
        ],
    )(lab, x, p, n)
    return out[0, 0]
